# Initial kernel scaffold; baseline (speedup 1.0000x reference)
import jax
import jax.numpy as jnp
from jax.experimental import pallas as pl


def kernel(x, edge_index, W1, a_src1, a_dst1, b1, W2, a_src2, a_dst2, b2):
    raise NotImplementedError("write your pallas kernel here")


# speedup vs baseline: 44.3616x; 44.3616x over previous
import functools

import jax
import jax.numpy as jnp
from jax import lax
from jax.experimental import pallas as pl
from jax.experimental.pallas import tpu as pltpu
from jax.experimental.pallas import tpu_sc as plsc

N = 10000
E = 320000
D = 128
H = 8
DH = 16
ACC = 136

NC = 2
NS = 16
NW = NC * NS
EPW = E // NW
CHUNK = 80
NCHUNK = EPW // CHUNK
NPAD = 10240
RPT = NPAD // NS
ZROWS = 128


def _edge_pass_body(tab_hbm, xw_hbm, src_hbm, dst_hbm, out_hbm,
                    srcv, dstv, sA, sD, rows, msg, zbuf, acc,
                    semA, semD, semR):
    c = lax.axis_index("c")
    s = lax.axis_index("s")
    wid = s * NC + c

    zeros16 = jnp.zeros((16,), jnp.float32)

    def zrow(r, carry):
        for k in range(ACC // 16):
            zbuf[r, pl.ds(16 * k, 16)] = zeros16
        return carry

    lax.fori_loop(0, ZROWS, zrow, 0)
    base_row = s * RPT
    for k in range(RPT // ZROWS):
        pltpu.sync_copy(zbuf, acc.at[pl.ds(base_row + ZROWS * k, ZROWS)])
    plsc.subcore_barrier()

    ebase = wid * EPW

    def chunk_body(g, carry):
        off = ebase + g * CHUNK
        pltpu.sync_copy(src_hbm.at[pl.ds(off, CHUNK)], srcv)
        pltpu.sync_copy(dst_hbm.at[pl.ds(off, CHUNK)], dstv)
        cpA = pltpu.async_copy(tab_hbm.at[srcv], sA, semA)
        cpD = pltpu.async_copy(tab_hbm.at[dstv], sD, semD)
        cpR = pltpu.async_copy(xw_hbm.at[srcv], rows, semR)
        cpA.wait()
        cpD.wait()

        def edge_body(i, ecarry):
            a16 = sA[i]
            d16 = sD[i]
            e16 = a16 + lax.rev(d16, (0,))
            e16 = jnp.maximum(e16, 0.2 * e16)
            ex = jnp.exp(e16)
            msg[i, pl.ds(0, 16)] = ex
            for h in range(H):
                msg[i, pl.ds(8 + 16 * h, 16)] = (
                    rows[i, pl.ds(16 * h, 16)] * ex[h])
            return ecarry

        cpR.wait()
        lax.fori_loop(0, CHUNK, edge_body, 0)
        pltpu.sync_copy(msg, acc.at[dstv], add=True)
        return carry

    lax.fori_loop(0, NCHUNK, chunk_body, 0)
    plsc.subcore_barrier()

    pltpu.sync_copy(acc.at[pl.ds(base_row, RPT)],
                    out_hbm.at[c, pl.ds(base_row, RPT)])


@functools.cache
def _edge_pass():
    return functools.partial(
        pl.kernel,
        out_type=jax.ShapeDtypeStruct((NC, NPAD, ACC), jnp.float32),
        mesh=plsc.VectorSubcoreMesh(core_axis_name="c", subcore_axis_name="s"),
        scratch_types=[
            pltpu.VMEM((CHUNK,), jnp.int32),
            pltpu.VMEM((CHUNK,), jnp.int32),
            pltpu.VMEM((CHUNK, 16), jnp.float32),
            pltpu.VMEM((CHUNK, 16), jnp.float32),
            pltpu.VMEM((CHUNK, D), jnp.float32),
            pltpu.VMEM((CHUNK, ACC), jnp.float32),
            pltpu.VMEM((ZROWS, ACC), jnp.float32),
            pltpu.VMEM_SHARED((NPAD, ACC), jnp.float32),
            pltpu.SemaphoreType.DMA,
            pltpu.SemaphoreType.DMA,
            pltpu.SemaphoreType.DMA,
        ],
        compiler_params=pltpu.CompilerParams(use_tc_tiling_on_sc=False),
    )(_edge_pass_body)


def _prep_body(x_ref, w_ref, ac_ref, xw_ref, tab_ref):
    xw = jnp.dot(x_ref[...], w_ref[...], preferred_element_type=jnp.float32)
    xw_ref[...] = xw
    tab_ref[...] = jnp.dot(xw, ac_ref[...], preferred_element_type=jnp.float32)


_prep = pl.pallas_call(
    _prep_body,
    out_shape=[jax.ShapeDtypeStruct((N, D), jnp.float32),
               jax.ShapeDtypeStruct((N, 16), jnp.float32)],
)


def _mid_body(a0_ref, a1_ref, p_ref, b_ref, w_ref, ac_ref, xw_ref, tab_ref):
    d = a0_ref[...][:N] + a1_ref[...][:N]
    den = jnp.dot(d[:, :H], p_ref[...],
                  preferred_element_type=jnp.float32) + 1e-16
    h1 = jnp.maximum(d[:, H:] / den + b_ref[...], 0.0)
    xw = jnp.dot(h1, w_ref[...], preferred_element_type=jnp.float32)
    xw_ref[...] = xw
    tab_ref[...] = jnp.dot(xw, ac_ref[...], preferred_element_type=jnp.float32)


_mid = pl.pallas_call(
    _mid_body,
    out_shape=[jax.ShapeDtypeStruct((N, D), jnp.float32),
               jax.ShapeDtypeStruct((N, 16), jnp.float32)],
)


def _final_body(a0_ref, a1_ref, p_ref, b_ref, out_ref):
    d = a0_ref[...][:N] + a1_ref[...][:N]
    den = jnp.dot(d[:, :H], p_ref[...],
                  preferred_element_type=jnp.float32) + 1e-16
    out_ref[...] = jnp.maximum(d[:, H:] / den + b_ref[...], 0.0)


_final = pl.pallas_call(
    _final_body,
    out_shape=jax.ShapeDtypeStruct((N, D), jnp.float32),
)


def _build_acomb(a_src, a_dst):
    A = jnp.zeros((D, 16), jnp.float32)
    for h in range(H):
        A = A.at[h * DH:(h + 1) * DH, h].set(a_src[h])
        A = A.at[h * DH:(h + 1) * DH, 15 - h].set(a_dst[h])
    return A


def _build_p8():
    P = jnp.zeros((H, D), jnp.float32)
    for h in range(H):
        P = P.at[h, h * DH:(h + 1) * DH].set(1.0)
    return P


def kernel(x, edge_index, W1, a_src1, a_dst1, b1, W2, a_src2, a_dst2, b2):
    src32 = edge_index[0].astype(jnp.int32)
    dst32 = edge_index[1].astype(jnp.int32)
    p8 = _build_p8()
    ac1 = _build_acomb(a_src1, a_dst1)
    ac2 = _build_acomb(a_src2, a_dst2)
    b1r = b1.reshape(1, D)
    b2r = b2.reshape(1, D)

    edge_pass = _edge_pass()
    xw1, tab1 = _prep(x, W1, ac1)
    acc1 = edge_pass(tab1, xw1, src32, dst32)
    xw2, tab2 = _mid(acc1[0], acc1[1], p8, b1r, W2, ac2)
    acc2 = edge_pass(tab2, xw2, src32, dst32)
    return _final(acc2[0], acc2[1], p8, b2r)


# speedup vs baseline: 52.0672x; 1.1737x over previous
import functools

import jax
import jax.numpy as jnp
from jax import lax
from jax.experimental import pallas as pl
from jax.experimental.pallas import tpu as pltpu
from jax.experimental.pallas import tpu_sc as plsc

N = 10000
E = 320000
D = 128
H = 8
DH = 16
ACC = 136
XP = D + 16

NC = 2
NS = 16
NW = NC * NS
EPW = E // NW
CHUNK = 80
NCHUNK = EPW // CHUNK
PAIRS = (NCHUNK - 1) // 2
NPAD = 10240
RPT = NPAD // NS


def _edge_pass_body(tab_hbm, xw_hbm, src_hbm, dst_hbm, out_hbm,
                    srcv0, dstv0, rows0, sA0, sD0,
                    srcv1, dstv1, rows1, sA1, sD1,
                    msg, acc, sem0, sem1):
    c = lax.axis_index("c")
    s = lax.axis_index("s")
    wid = s * NC + c

    zeros16 = jnp.zeros((16,), jnp.float32)

    def zrow(r, carry):
        for k in range(ACC // 16):
            msg[r, pl.ds(16 * k, 16)] = zeros16
        msg[r, pl.ds(ACC - 16, 16)] = zeros16
        return carry

    lax.fori_loop(0, CHUNK, zrow, 0)
    base_row = s * RPT
    for k in range(RPT // CHUNK):
        pltpu.sync_copy(msg, acc.at[pl.ds(base_row + CHUNK * k, CHUNK)])
    plsc.subcore_barrier()

    ebase = wid * EPW

    def issue(g, srcv, dstv, rows, sA, sD, sem):
        off = ebase + g * CHUNK
        pltpu.sync_copy(src_hbm.at[pl.ds(off, CHUNK)], srcv)
        pltpu.sync_copy(dst_hbm.at[pl.ds(off, CHUNK)], dstv)
        cpR = pltpu.async_copy(xw_hbm.at[srcv], rows, sem)
        cpA = pltpu.async_copy(tab_hbm.at[srcv], sA, sem)
        cpD = pltpu.async_copy(tab_hbm.at[dstv], sD, sem)
        return cpR, cpA, cpD

    def drain(srcv, dstv, rows, sA, sD, sem):
        pltpu.make_async_copy(xw_hbm.at[srcv], rows, sem).wait()
        pltpu.make_async_copy(tab_hbm.at[srcv], sA, sem).wait()
        pltpu.make_async_copy(tab_hbm.at[dstv], sD, sem).wait()

    def compute_scatter(dstv, rows, sA, sD):
        def edge_body(i, ecarry):
            a16 = sA[i]
            d16 = sD[i]
            e16 = a16 + lax.rev(d16, (0,))
            e16 = jnp.maximum(e16, 0.2 * e16)
            ex = jnp.exp(e16)
            msg[i, pl.ds(0, 16)] = ex
            for h in range(H):
                msg[i, pl.ds(8 + 16 * h, 16)] = (
                    rows[i, pl.ds(16 * h, 16)] * ex[h])
            return ecarry

        lax.fori_loop(0, CHUNK, edge_body, 0)
        pltpu.sync_copy(msg, acc.at[dstv], add=True)

    issue(0, srcv0, dstv0, rows0, sA0, sD0, sem0)

    def pair_body(p, carry):
        g0 = 2 * p
        drain(srcv0, dstv0, rows0, sA0, sD0, sem0)
        cps = issue(g0 + 1, srcv1, dstv1, rows1, sA1, sD1, sem1)
        compute_scatter(dstv0, rows0, sA0, sD0)
        issue(g0 + 2, srcv0, dstv0, rows0, sA0, sD0, sem0)
        for cp in cps:
            cp.wait()
        compute_scatter(dstv1, rows1, sA1, sD1)
        return carry

    lax.fori_loop(0, PAIRS, pair_body, 0)
    drain(srcv0, dstv0, rows0, sA0, sD0, sem0)
    compute_scatter(dstv0, rows0, sA0, sD0)
    plsc.subcore_barrier()

    pltpu.sync_copy(acc.at[pl.ds(base_row, RPT)],
                    out_hbm.at[c, pl.ds(base_row, RPT)])


@functools.cache
def _edge_pass():
    return functools.partial(
        pl.kernel,
        out_type=jax.ShapeDtypeStruct((NC, NPAD, ACC), jnp.float32),
        mesh=plsc.VectorSubcoreMesh(core_axis_name="c", subcore_axis_name="s"),
        scratch_types=[
            pltpu.VMEM((CHUNK,), jnp.int32),
            pltpu.VMEM((CHUNK,), jnp.int32),
            pltpu.VMEM((CHUNK, D), jnp.float32),
            pltpu.VMEM((CHUNK, 16), jnp.float32),
            pltpu.VMEM((CHUNK, 16), jnp.float32),
            pltpu.VMEM((CHUNK,), jnp.int32),
            pltpu.VMEM((CHUNK,), jnp.int32),
            pltpu.VMEM((CHUNK, D), jnp.float32),
            pltpu.VMEM((CHUNK, 16), jnp.float32),
            pltpu.VMEM((CHUNK, 16), jnp.float32),
            pltpu.VMEM((CHUNK, ACC), jnp.float32),
            pltpu.VMEM_SHARED((NPAD, ACC), jnp.float32),
            pltpu.SemaphoreType.DMA,
            pltpu.SemaphoreType.DMA,
        ],
        compiler_params=pltpu.CompilerParams(use_tc_tiling_on_sc=False),
    )(_edge_pass_body)


def _prep_body(x_ref, w_ref, ac_ref, xw_ref, tab_ref):
    xw = jnp.dot(x_ref[...], w_ref[...], preferred_element_type=jnp.float32)
    xw_ref[...] = xw
    tab_ref[...] = jnp.dot(xw, ac_ref[...], preferred_element_type=jnp.float32)


_prep = pl.pallas_call(
    _prep_body,
    out_shape=[jax.ShapeDtypeStruct((N, D), jnp.float32),
               jax.ShapeDtypeStruct((N, 16), jnp.float32)],
)


def _mid_body(a0_ref, a1_ref, p_ref, b_ref, w_ref, ac_ref, xw_ref, tab_ref):
    d = a0_ref[...][:N] + a1_ref[...][:N]
    den = jnp.dot(d[:, :H], p_ref[...],
                  preferred_element_type=jnp.float32) + 1e-16
    h1 = jnp.maximum(d[:, H:] / den + b_ref[...], 0.0)
    xw = jnp.dot(h1, w_ref[...], preferred_element_type=jnp.float32)
    xw_ref[...] = xw
    tab_ref[...] = jnp.dot(xw, ac_ref[...], preferred_element_type=jnp.float32)


_mid = pl.pallas_call(
    _mid_body,
    out_shape=[jax.ShapeDtypeStruct((N, D), jnp.float32),
               jax.ShapeDtypeStruct((N, 16), jnp.float32)],
)


def _final_body(a0_ref, a1_ref, p_ref, b_ref, out_ref):
    d = a0_ref[...][:N] + a1_ref[...][:N]
    den = jnp.dot(d[:, :H], p_ref[...],
                  preferred_element_type=jnp.float32) + 1e-16
    out_ref[...] = jnp.maximum(d[:, H:] / den + b_ref[...], 0.0)


_final = pl.pallas_call(
    _final_body,
    out_shape=jax.ShapeDtypeStruct((N, D), jnp.float32),
)


def _build_acomb(a_src, a_dst):
    A = jnp.zeros((D, 16), jnp.float32)
    for h in range(H):
        A = A.at[h * DH:(h + 1) * DH, h].set(a_src[h])
        A = A.at[h * DH:(h + 1) * DH, 15 - h].set(a_dst[h])
    return A


def _build_p8():
    P = jnp.zeros((H, D), jnp.float32)
    for h in range(H):
        P = P.at[h, h * DH:(h + 1) * DH].set(1.0)
    return P


def kernel(x, edge_index, W1, a_src1, a_dst1, b1, W2, a_src2, a_dst2, b2):
    src32 = edge_index[0].astype(jnp.int32)
    dst32 = edge_index[1].astype(jnp.int32)
    p8 = _build_p8()
    ac1 = _build_acomb(a_src1, a_dst1)
    ac2 = _build_acomb(a_src2, a_dst2)
    b1r = b1.reshape(1, D)
    b2r = b2.reshape(1, D)

    edge_pass = _edge_pass()
    xw1, tab1 = _prep(x, W1, ac1)
    acc1 = edge_pass(tab1, xw1, src32, dst32)
    xw2, tab2 = _mid(acc1[0], acc1[1], p8, b1r, W2, ac2)
    acc2 = edge_pass(tab2, xw2, src32, dst32)
    return _final(acc2[0], acc2[1], p8, b2r)
